# baseline (speedup 1.0000x reference)
import numpy as np
import jax
import jax.numpy as jnp
from jax import lax
from jax.experimental import pallas as pl
from jax.experimental.pallas import tpu as pltpu

ROWS = 128
COLS = 100000
BR = 8

_ROT0 = (13, 15, 26, 6)
_ROT1 = (17, 29, 16, 24)


def _rotl(x, r):
    return lax.shift_left(x, np.uint32(r)) | lax.shift_right_logical(
        x, np.uint32(32 - r))


def _rounds(x0, x1, rots):
    for r in rots:
        x0 = x0 + x1
        x1 = _rotl(x1, r)
        x1 = x0 ^ x1
    return x0, x1


def _threefry_bits(i):
    k0 = jnp.uint32(0)
    k1 = jnp.uint32(42)
    ks2 = k0 ^ k1 ^ jnp.uint32(0x1BD11BDA)
    x0 = jnp.zeros_like(i) + k0
    x1 = i + k1
    x0, x1 = _rounds(x0, x1, _ROT0)
    x0 = x0 + k1
    x1 = x1 + ks2 + jnp.uint32(1)
    x0, x1 = _rounds(x0, x1, _ROT1)
    x0 = x0 + ks2
    x1 = x1 + k0 + jnp.uint32(2)
    x0, x1 = _rounds(x0, x1, _ROT0)
    x0 = x0 + k0
    x1 = x1 + k1 + jnp.uint32(3)
    x0, x1 = _rounds(x0, x1, _ROT1)
    x0 = x0 + k1
    x1 = x1 + ks2 + jnp.uint32(4)
    x0, x1 = _rounds(x0, x1, _ROT0)
    x0 = x0 + ks2
    x1 = x1 + k0 + jnp.uint32(5)
    return x0 ^ x1


def _gumbel_softmax_body(x_ref, o_ref):
    br, cols = x_ref.shape
    base = lax.convert_element_type(pl.program_id(0) * br, jnp.uint32)
    row = lax.broadcasted_iota(jnp.uint32, (br, cols), 0) + base
    col = lax.broadcasted_iota(jnp.uint32, (br, cols), 1)
    idx = row * jnp.uint32(cols) + col

    bits = _threefry_bits(idx)
    float_bits = lax.shift_right_logical(bits, np.uint32(9)) | jnp.uint32(
        0x3F800000)
    f = lax.bitcast_convert_type(float_bits, jnp.float32) - jnp.float32(1.0)
    tiny = jnp.float32(np.finfo(np.float32).tiny)
    u = jnp.maximum(tiny, f + tiny)
    g = -jnp.log(-jnp.log(u))

    y = x_ref[...] + g
    m = jnp.max(y, axis=1, keepdims=True)
    e = jnp.exp(y - m)
    s = jnp.sum(e, axis=1, keepdims=True)
    o_ref[...] = e / s


def kernel(logits):
    return pl.pallas_call(
        _gumbel_softmax_body,
        grid=(ROWS // BR,),
        in_specs=[pl.BlockSpec((BR, COLS), lambda i: (i, 0))],
        out_specs=pl.BlockSpec((BR, COLS), lambda i: (i, 0)),
        out_shape=jax.ShapeDtypeStruct((ROWS, COLS), jnp.float32),
        compiler_params=pltpu.CompilerParams(
            dimension_semantics=("parallel",)),
    )(logits)
